# baseline (speedup 1.0000x reference)
import jax
import jax.numpy as jnp
from jax.experimental import pallas as pl

SEQ_TILE = 2048


def _add_pos_kernel(x_ref, pos_ref, o_ref):
    o_ref[...] = x_ref[...] + pos_ref[...]


def _add_pos_bcast_kernel(x_ref, pos_ref, o_ref):
    o_ref[...] = x_ref[...] + pos_ref[...][None]


def kernel(x, pos_table):
    batch, seq, embed = x.shape
    positions = pos_table[:seq]
    tile = 512
    n_seq_tiles = seq // tile

    return pl.pallas_call(
        _add_pos_bcast_kernel,
        grid=(n_seq_tiles,),
        in_specs=[
            pl.BlockSpec((batch, tile, embed), lambda i: (0, i, 0)),
            pl.BlockSpec((tile, embed), lambda i: (i, 0)),
        ],
        out_specs=pl.BlockSpec((batch, tile, embed), lambda i: (0, i, 0)),
        out_shape=jax.ShapeDtypeStruct(x.shape, x.dtype),
    )(x, positions)


# speedup vs baseline: 1.0242x; 1.0242x over previous
import jax
import jax.numpy as jnp
from jax.experimental import pallas as pl

ROW_TILE = 2048


def _add_pos_kernel(x_ref, pos_ref, o_ref):
    seq = pos_ref.shape[0]
    off = (pl.program_id(0) * ROW_TILE) % seq
    o_ref[...] = x_ref[...] + pos_ref[pl.ds(off, ROW_TILE), :]


def kernel(x, pos_table):
    batch, seq, embed = x.shape
    positions = pos_table[:seq]
    xf = x.reshape(batch * seq, embed)
    n_tiles = (batch * seq) // ROW_TILE

    out = pl.pallas_call(
        _add_pos_kernel,
        grid=(n_tiles,),
        in_specs=[
            pl.BlockSpec((ROW_TILE, embed), lambda i: (i, 0)),
            pl.BlockSpec((seq, embed), lambda i: (0, 0)),
        ],
        out_specs=pl.BlockSpec((ROW_TILE, embed), lambda i: (i, 0)),
        out_shape=jax.ShapeDtypeStruct(xf.shape, x.dtype),
    )(xf, positions)
    return out.reshape(x.shape)


# speedup vs baseline: 1.0398x; 1.0152x over previous
import jax
import jax.numpy as jnp
from jax import lax
from jax.experimental import pallas as pl
from jax.experimental.pallas import tpu as pltpu

TILE = 1024
DEPTH = 4


def _pipeline_kernel(x_hbm, pos_hbm, o_hbm, xbuf, obuf, pbuf, in_sems, out_sems, pos_sem):
    n_rows = x_hbm.shape[0]
    seq = pos_hbm.shape[0]
    n_tiles = n_rows // TILE

    pos_copy = pltpu.make_async_copy(pos_hbm, pbuf, pos_sem)
    pos_copy.start()

    for k in range(DEPTH):
        pltpu.make_async_copy(
            x_hbm.at[pl.ds(k * TILE, TILE), :], xbuf.at[k], in_sems.at[k]
        ).start()

    pos_copy.wait()

    def step(t, carry):
        slot = lax.rem(t, DEPTH)
        pltpu.make_async_copy(
            x_hbm.at[pl.ds(t * TILE, TILE), :], xbuf.at[slot], in_sems.at[slot]
        ).wait()

        @pl.when(t >= DEPTH)
        def _():
            pltpu.make_async_copy(
                obuf.at[slot], o_hbm.at[pl.ds((t - DEPTH) * TILE, TILE), :],
                out_sems.at[slot],
            ).wait()

        off = lax.rem(t * TILE, seq)
        obuf[slot] = xbuf[slot] + pbuf[pl.ds(off, TILE), :]

        pltpu.make_async_copy(
            obuf.at[slot], o_hbm.at[pl.ds(t * TILE, TILE), :], out_sems.at[slot]
        ).start()

        @pl.when(t + DEPTH < n_tiles)
        def _():
            pltpu.make_async_copy(
                x_hbm.at[pl.ds((t + DEPTH) * TILE, TILE), :], xbuf.at[slot],
                in_sems.at[slot],
            ).start()

        return carry

    lax.fori_loop(0, n_tiles, step, 0)

    for k in range(n_tiles - DEPTH, n_tiles):
        slot = k % DEPTH
        pltpu.make_async_copy(
            obuf.at[slot], o_hbm.at[pl.ds(k * TILE, TILE), :], out_sems.at[slot]
        ).wait()


def kernel(x, pos_table):
    batch, seq, embed = x.shape
    positions = pos_table[:seq]
    xf = x.reshape(batch * seq, embed)

    out = pl.pallas_call(
        _pipeline_kernel,
        in_specs=[
            pl.BlockSpec(memory_space=pl.ANY),
            pl.BlockSpec(memory_space=pl.ANY),
        ],
        out_specs=pl.BlockSpec(memory_space=pl.ANY),
        out_shape=jax.ShapeDtypeStruct(xf.shape, x.dtype),
        scratch_shapes=[
            pltpu.MemorySpace.VMEM((DEPTH, TILE, embed), jnp.float32),
            pltpu.MemorySpace.VMEM((DEPTH, TILE, embed), jnp.float32),
            pltpu.VMEM((seq, embed), jnp.float32),
            pltpu.SemaphoreType.DMA((DEPTH,)),
            pltpu.SemaphoreType.DMA((DEPTH,)),
            pltpu.SemaphoreType.DMA,
        ],
    )(xf, positions)
    return out.reshape(x.shape)
